# baseline (speedup 1.0000x reference)
import functools

import jax
import jax.numpy as jnp
from jax import lax
from jax.experimental import pallas as pl
from jax.experimental.pallas import tpu as pltpu
from jax.experimental.pallas import tpu_sc as plsc

_SC_SUBCORES = 16
_SC_LANES = 16
_N_HI = 64


def _sc_gather(emb, n_idx):
    d = emb.shape[1]
    mesh = plsc.VectorSubcoreMesh(
        core_axis_name="c", subcore_axis_name="s", num_cores=1
    )
    b_per_w = n_idx // _SC_SUBCORES

    @functools.partial(
        pl.kernel,
        mesh=mesh,
        out_type=jax.ShapeDtypeStruct((n_idx, d), emb.dtype),
        scratch_types=[
            pltpu.VMEM((b_per_w,), jnp.int32),
            pltpu.VMEM((b_per_w, d), jnp.float32),
            pltpu.SemaphoreType.DMA,
        ],
    )
    def gather_kernel(table_hbm, out_hbm, idx_v, rows_v, sem):
        wid = lax.axis_index("s")
        base = wid * b_per_w
        @pl.loop(0, b_per_w, step=_SC_LANES)
        def _(c):
            idx_v[pl.ds(c, _SC_LANES)] = (
                lax.iota(jnp.int32, _SC_LANES) + base + c
            )
        pltpu.async_copy(table_hbm.at[idx_v], rows_v, sem).wait()
        pltpu.sync_copy(rows_v, out_hbm.at[pl.ds(base, b_per_w)])

    return gather_kernel(emb)


def _add_body(x_ref, e_ref, o_ref):
    o_ref[...] = x_ref[...] + e_ref[...][None, :, None, :]


def _add_body_alias(x_ref, e_ref, _alias_ref, o_ref):
    o_ref[...] = x_ref[...] + e_ref[...][None, :, None, :]


def _tc_add_lo(x, emb, n_lo, bn):
    B, N, T, D = x.shape
    return pl.pallas_call(
        _add_body,
        grid=(n_lo // bn, B),
        in_specs=[
            pl.BlockSpec((1, bn, T, D), lambda j, i: (i, j, 0, 0)),
            pl.BlockSpec((bn, D), lambda j, i: (j, 0)),
        ],
        out_specs=pl.BlockSpec((1, bn, T, D), lambda j, i: (i, j, 0, 0)),
        out_shape=jax.ShapeDtypeStruct(x.shape, x.dtype),
        compiler_params=pltpu.CompilerParams(
            dimension_semantics=("parallel", "parallel")),
    )(x, emb)


def _tc_add_hi(x, pos, partial, n_lo, bn):
    B, N, T, D = x.shape
    n_hi = N - n_lo
    blk0 = n_lo // bn
    return pl.pallas_call(
        _add_body_alias,
        grid=(n_hi // bn, B),
        in_specs=[
            pl.BlockSpec((1, bn, T, D), lambda j, i: (i, j + blk0, 0, 0)),
            pl.BlockSpec((bn, D), lambda j, i: (j + blk0, 0)),
            pl.BlockSpec(memory_space=pl.ANY),
        ],
        out_specs=pl.BlockSpec(
            (1, bn, T, D), lambda j, i: (i, j + blk0, 0, 0)
        ),
        out_shape=jax.ShapeDtypeStruct(x.shape, x.dtype),
        input_output_aliases={2: 0},
        compiler_params=pltpu.CompilerParams(
            dimension_semantics=("parallel", "parallel")),
    )(x, pos, partial)


def kernel(x, emb):
    n = x.shape[1]
    n_lo = n - _N_HI
    partial = _tc_add_lo(x, emb, n_lo, bn=n_lo // 4)
    pos = _sc_gather(emb, n)
    return _tc_add_hi(x, pos, partial, n_lo, bn=_N_HI)


# speedup vs baseline: 1.0276x; 1.0276x over previous
import functools

import jax
import jax.numpy as jnp
from jax import lax
from jax.experimental import pallas as pl
from jax.experimental.pallas import tpu as pltpu
from jax.experimental.pallas import tpu_sc as plsc

_SC_SUBCORES = 16
_SC_LANES = 16
_N_HI = 64


def _sc_gather(emb, n_idx):
    d = emb.shape[1]
    mesh = plsc.VectorSubcoreMesh(
        core_axis_name="c", subcore_axis_name="s", num_cores=1
    )
    b_per_w = n_idx // _SC_SUBCORES

    @functools.partial(
        pl.kernel,
        mesh=mesh,
        out_type=jax.ShapeDtypeStruct((n_idx, d), emb.dtype),
        scratch_types=[
            pltpu.VMEM((b_per_w,), jnp.int32),
            pltpu.VMEM((b_per_w, d), jnp.float32),
            pltpu.SemaphoreType.DMA,
        ],
    )
    def gather_kernel(table_hbm, out_hbm, idx_v, rows_v, sem):
        wid = lax.axis_index("s")
        base = wid * b_per_w
        @pl.loop(0, b_per_w, step=_SC_LANES)
        def _(c):
            idx_v[pl.ds(c, _SC_LANES)] = (
                lax.iota(jnp.int32, _SC_LANES) + base + c
            )
        pltpu.async_copy(table_hbm.at[idx_v], rows_v, sem).wait()
        pltpu.sync_copy(rows_v, out_hbm.at[pl.ds(base, b_per_w)])

    return gather_kernel(emb)


def _add_body(x_ref, e_ref, o_ref):
    o_ref[...] = x_ref[...] + e_ref[...][None, :, None, :]


def _add_body_alias(x_ref, e_ref, _alias_ref, o_ref):
    o_ref[...] = x_ref[...] + e_ref[...][None, :, None, :]


def _tc_add_lo(x, emb, n_lo, bn):
    B, N, T, D = x.shape
    return pl.pallas_call(
        _add_body,
        grid=(n_lo // bn, B),
        in_specs=[
            pl.BlockSpec((1, bn, T, D), lambda j, i: (i, j, 0, 0)),
            pl.BlockSpec((bn, D), lambda j, i: (j, 0)),
        ],
        out_specs=pl.BlockSpec((1, bn, T, D), lambda j, i: (i, j, 0, 0)),
        out_shape=jax.ShapeDtypeStruct(x.shape, x.dtype),
        compiler_params=pltpu.CompilerParams(
            dimension_semantics=("parallel", "parallel")),
    )(x, emb)


def _tc_add_hi(x, pos, partial, n_lo, bn):
    B, N, T, D = x.shape
    n_hi = N - n_lo
    blk0 = n_lo // bn
    return pl.pallas_call(
        _add_body_alias,
        grid=(n_hi // bn, B),
        in_specs=[
            pl.BlockSpec((1, bn, T, D), lambda j, i: (i, j + blk0, 0, 0)),
            pl.BlockSpec((bn, D), lambda j, i: (j + blk0, 0)),
            pl.BlockSpec(memory_space=pl.ANY),
        ],
        out_specs=pl.BlockSpec(
            (1, bn, T, D), lambda j, i: (i, j + blk0, 0, 0)
        ),
        out_shape=jax.ShapeDtypeStruct(x.shape, x.dtype),
        input_output_aliases={2: 0},
        compiler_params=pltpu.CompilerParams(
            dimension_semantics=("parallel", "parallel")),
    )(x, pos, partial)


def kernel(x, emb):
    n = x.shape[1]
    n_lo = n - _N_HI
    partial = _tc_add_lo(x, emb, n_lo, bn=n_lo // 2)
    pos = _sc_gather(emb, n)
    return _tc_add_hi(x, pos, partial, n_lo, bn=_N_HI)


# speedup vs baseline: 1.0349x; 1.0071x over previous
import functools

import jax
import jax.numpy as jnp
from jax import lax
from jax.experimental import pallas as pl
from jax.experimental.pallas import tpu as pltpu
from jax.experimental.pallas import tpu_sc as plsc

_SC_SUBCORES = 16
_SC_LANES = 16
_N_HI = 16


def _sc_gather(emb, n_idx):
    d = emb.shape[1]
    mesh = plsc.VectorSubcoreMesh(
        core_axis_name="c", subcore_axis_name="s", num_cores=1
    )
    b_per_w = n_idx // _SC_SUBCORES

    @functools.partial(
        pl.kernel,
        mesh=mesh,
        out_type=jax.ShapeDtypeStruct((n_idx, d), emb.dtype),
        scratch_types=[
            pltpu.VMEM((b_per_w,), jnp.int32),
            pltpu.VMEM((b_per_w, d), jnp.float32),
            pltpu.SemaphoreType.DMA,
        ],
    )
    def gather_kernel(table_hbm, out_hbm, idx_v, rows_v, sem):
        wid = lax.axis_index("s")
        base = wid * b_per_w
        @pl.loop(0, b_per_w, step=_SC_LANES)
        def _(c):
            idx_v[pl.ds(c, _SC_LANES)] = (
                lax.iota(jnp.int32, _SC_LANES) + base + c
            )
        pltpu.async_copy(table_hbm.at[idx_v], rows_v, sem).wait()
        pltpu.sync_copy(rows_v, out_hbm.at[pl.ds(base, b_per_w)])

    return gather_kernel(emb)


def _add_body(x_ref, e_ref, o_ref):
    o_ref[...] = x_ref[...] + e_ref[...][None, :, None, :]


def _add_body_alias(x_ref, e_ref, _alias_ref, o_ref):
    o_ref[...] = x_ref[...] + e_ref[...][None, :, None, :]


def _tc_add_lo(x, emb, n_lo, bn):
    B, N, T, D = x.shape
    return pl.pallas_call(
        _add_body,
        grid=(n_lo // bn, B),
        in_specs=[
            pl.BlockSpec((1, bn, T, D), lambda j, i: (i, j, 0, 0)),
            pl.BlockSpec((bn, D), lambda j, i: (j, 0)),
        ],
        out_specs=pl.BlockSpec((1, bn, T, D), lambda j, i: (i, j, 0, 0)),
        out_shape=jax.ShapeDtypeStruct(x.shape, x.dtype),
        compiler_params=pltpu.CompilerParams(
            dimension_semantics=("parallel", "parallel")),
    )(x, emb)


def _tc_add_hi(x, pos, partial, n_lo, bn):
    B, N, T, D = x.shape
    n_hi = N - n_lo
    blk0 = n_lo // bn
    return pl.pallas_call(
        _add_body_alias,
        grid=(n_hi // bn, B),
        in_specs=[
            pl.BlockSpec((1, bn, T, D), lambda j, i: (i, j + blk0, 0, 0)),
            pl.BlockSpec((bn, D), lambda j, i: (j + blk0, 0)),
            pl.BlockSpec(memory_space=pl.ANY),
        ],
        out_specs=pl.BlockSpec(
            (1, bn, T, D), lambda j, i: (i, j + blk0, 0, 0)
        ),
        out_shape=jax.ShapeDtypeStruct(x.shape, x.dtype),
        input_output_aliases={2: 0},
        compiler_params=pltpu.CompilerParams(
            dimension_semantics=("parallel", "parallel")),
    )(x, pos, partial)


def kernel(x, emb):
    n = x.shape[1]
    n_lo = n - _N_HI
    partial = _tc_add_lo(x, emb, n_lo, bn=n_lo // 2)
    pos = _sc_gather(emb, n)
    return _tc_add_hi(x, pos, partial, n_lo, bn=_N_HI)
